# Initial kernel scaffold; baseline (speedup 1.0000x reference)
import jax
import jax.numpy as jnp
from jax.experimental import pallas as pl


def kernel(entity_emb, user_emb, latent_emb, edge_index, edge_type, interact_mat, weight, entity_cate_set, w1_w, w1_b, w2_w, w2_b, ua_w, ua_b, wa_w, wa_b):
    raise NotImplementedError("write your pallas kernel here")


# speedup vs baseline: 5.5909x; 5.5909x over previous
import jax
import jax.numpy as jnp
from jax import lax
from jax.experimental import pallas as pl
from jax.experimental.pallas import tpu as pltpu
from jax.experimental.pallas import tpu_sc as plsc

N_ENT = 10000
N_ENT_PAD = 10240
EMB = 64
N_EDGES = 640000
CNTW = 16

NC, NS = 2, 16
NW = NC * NS
E1 = 50
K = 8
C = E1 * K
ROWS_TOT = N_EDGES // E1
ROWS_PT = ROWS_TOT // NW
NCHUNK = ROWS_PT // K
STRIPE = N_ENT_PAD // NS


def _sc_body(tail_hbm, head_hbm, et_hbm, emb_hbm, wrel_hbm,
             acc_out, cnt_out,
             tidx, hidx, eidx, rows, wrows, ones, zb,
             acc_sh, cnt_sh, sem_a, sem_b):
    cid = lax.axis_index("c")
    sid = lax.axis_index("s")
    wid = cid * NS + sid
    row0 = wid * ROWS_PT

    z16 = jnp.zeros((16,), jnp.float32)
    o16 = jnp.ones((16,), jnp.float32)

    def _zb(i, c):
        zb[i, :] = z16
        return c
    lax.fori_loop(0, STRIPE, _zb, 0)

    def _zr(i, c):
        for k in range(EMB // 16):
            rows[0, i, pl.ds(k * 16, 16)] = z16
        return c
    lax.fori_loop(0, E1, _zr, 0)

    s0 = sid * STRIPE
    def _za(i, c):
        pltpu.sync_copy(rows.at[0], acc_sh.at[pl.ds(s0 + i * E1, E1)])
        return c
    lax.fori_loop(0, STRIPE // E1, _za, 0)
    rem = STRIPE % E1
    pltpu.sync_copy(rows.at[0, pl.ds(0, rem)],
                    acc_sh.at[pl.ds(s0 + (STRIPE // E1) * E1, rem)])
    pltpu.sync_copy(zb, cnt_sh.at[pl.ds(s0, STRIPE)])

    def _ob(i, c):
        ones[i, :] = o16
        return c
    lax.fori_loop(0, E1, _ob, 0)

    plsc.subcore_barrier()

    def _chunk(i, c):
        r = row0 + i * K
        pltpu.sync_copy(tail_hbm.at[pl.ds(r, K)], tidx)
        pltpu.sync_copy(et_hbm.at[pl.ds(r, K)], eidx)
        pltpu.sync_copy(head_hbm.at[pl.ds(r, K)], hidx)
        cps = []
        for j in range(K):
            cps.append(pltpu.async_copy(emb_hbm.at[tidx.at[j]], rows.at[j], sem_a))
            cps.append(pltpu.async_copy(wrel_hbm.at[eidx.at[j]], wrows.at[j], sem_b))
        for cp in cps:
            cp.wait()
        for j in range(K):
            def _mul(e, cc):
                for k in range(EMB // 16):
                    s = pl.ds(k * 16, 16)
                    rows[j, e, s] = rows[j, e, s] * wrows[j, e, s]
                return cc
            lax.fori_loop(0, E1, _mul, 0)
        for j in range(K):
            pltpu.sync_copy(rows.at[j], acc_sh.at[hidx.at[j]], add=True)
            pltpu.sync_copy(ones, cnt_sh.at[hidx.at[j]], add=True)
        return c
    lax.fori_loop(0, NCHUNK, _chunk, 0)

    plsc.subcore_barrier()

    pltpu.sync_copy(acc_sh.at[pl.ds(s0, STRIPE)], acc_out.at[cid, pl.ds(s0, STRIPE)])
    pltpu.sync_copy(cnt_sh.at[pl.ds(s0, STRIPE)], cnt_out.at[cid, pl.ds(s0, STRIPE)])


_sc_agg = pl.kernel(
    _sc_body,
    out_type=(
        pltpu.HBM((NC, N_ENT_PAD, EMB), jnp.float32),
        pltpu.HBM((NC, N_ENT_PAD, CNTW), jnp.float32),
    ),
    mesh=plsc.VectorSubcoreMesh(core_axis_name="c", subcore_axis_name="s"),
    compiler_params=pltpu.CompilerParams(use_tc_tiling_on_sc=False),
    scratch_types=[
        pltpu.VMEM((K, E1), jnp.int32),
        pltpu.VMEM((K, E1), jnp.int32),
        pltpu.VMEM((K, E1), jnp.int32),
        pltpu.VMEM((K, E1, EMB), jnp.float32),
        pltpu.VMEM((K, E1, EMB), jnp.float32),
        pltpu.VMEM((E1, CNTW), jnp.float32),
        pltpu.VMEM((STRIPE, CNTW), jnp.float32),
        pltpu.VMEM_SHARED((N_ENT_PAD, EMB), jnp.float32),
        pltpu.VMEM_SHARED((N_ENT_PAD, CNTW), jnp.float32),
        pltpu.SemaphoreType.DMA,
        pltpu.SemaphoreType.DMA,
    ],
)


def _combine_body(acc_ref, cnt_ref, out_ref):
    a = acc_ref[0] + acc_ref[1]
    c = cnt_ref[0] + cnt_ref[1]
    cnt = jnp.maximum(c[:, 0:1], 1.0)
    out_ref[...] = a / cnt


def _leaky(x):
    return jnp.where(x >= 0, x, 0.2 * x)


def _user_body(im_ref, emb_ref, u_ref, lat_ref, w_ref,
               w1w_ref, w1b_ref, w2w_ref, w2b_ref,
               uaw_ref, uab_ref, waw_ref, wab_ref,
               uout_ref, lat_out_ref):
    f32 = jnp.float32
    def dott(a, b):
        return lax.dot_general(a, b, (((1,), (1,)), ((), ())),
                               preferred_element_type=f32)

    ua = jnp.dot(im_ref[...], emb_ref[...], preferred_element_type=f32)

    w1w, w1b = w1w_ref[...], w1b_ref[...]
    w2w, w2b = w2w_ref[...], w2b_ref[...]
    lat = lat_ref[...]
    w = w_ref[...]

    u1 = dott(u_ref[...], w1w) + w1b
    l1 = dott(lat, w1w) + w1b
    s2 = _leaky(dott(dott(u1, l1), uaw_ref[...]) + uab_ref[...])
    m = jnp.max(s2, axis=1, keepdims=True)
    e = jnp.exp(s2 - m)
    score = e / jnp.sum(e, axis=1, keepdims=True)

    l2 = dott(lat, w2w) + w2b
    wt2 = dott(w, w2w) + w2b
    s3 = _leaky(dott(dott(l2, wt2), waw_ref[...]) + wab_ref[...])
    m3 = jnp.max(s3, axis=1, keepdims=True)
    e3 = jnp.exp(s3 - m3)
    sm3 = e3 / jnp.sum(e3, axis=1, keepdims=True)
    latent_new = jnp.dot(sm3, w, preferred_element_type=f32)

    gate = 1.0 + jnp.dot(score, latent_new, preferred_element_type=f32)
    uout_ref[...] = ua * gate
    lat_out_ref[...] = latent_new


def kernel(entity_emb, user_emb, latent_emb, edge_index, edge_type, interact_mat,
           weight, entity_cate_set, w1_w, w1_b, w2_w, w2_b, ua_w, ua_b, wa_w, wa_b):
    n_users = user_emb.shape[0]
    n_rel1 = weight.shape[0]
    n_fac = latent_emb.shape[0]

    head = edge_index[0].reshape(ROWS_TOT, E1)
    tail = edge_index[1].reshape(ROWS_TOT, E1)
    et0 = (edge_type - 1).reshape(ROWS_TOT, E1)

    acc, cnt = _sc_agg(tail, head, et0, entity_emb, weight)

    entity_agg_pad = pl.pallas_call(
        _combine_body,
        out_shape=jax.ShapeDtypeStruct((N_ENT_PAD, EMB), jnp.float32),
    )(acc, cnt)
    entity_agg = entity_agg_pad[:N_ENT]

    BU = 128
    grid = (n_users // BU,)
    full = lambda s: pl.BlockSpec(s, lambda i: (0, 0))
    user_agg, latent_new = pl.pallas_call(
        _user_body,
        grid=grid,
        in_specs=[
            pl.BlockSpec((BU, N_ENT), lambda i: (i, 0)),
            full((N_ENT, EMB)),
            pl.BlockSpec((BU, EMB), lambda i: (i, 0)),
            full((n_fac, EMB)),
            full((n_rel1, EMB)),
            full(w1_w.shape),
            full((1, EMB)),
            full(w2_w.shape),
            full((1, EMB)),
            full(ua_w.shape),
            full((1, n_fac)),
            full(wa_w.shape),
            full((1, n_rel1)),
        ],
        out_specs=[
            pl.BlockSpec((BU, EMB), lambda i: (i, 0)),
            pl.BlockSpec((n_fac, EMB), lambda i: (0, 0)),
        ],
        out_shape=[
            jax.ShapeDtypeStruct((n_users, EMB), jnp.float32),
            jax.ShapeDtypeStruct((n_fac, EMB), jnp.float32),
        ],
    )(interact_mat, entity_emb, user_emb, latent_emb, weight,
      w1_w, w1_b.reshape(1, EMB), w2_w, w2_b.reshape(1, EMB),
      ua_w, ua_b.reshape(1, n_fac), wa_w, wa_b.reshape(1, n_rel1))

    return entity_agg, user_agg, latent_new
